# Initial kernel scaffold; baseline (speedup 1.0000x reference)
import jax
import jax.numpy as jnp
from jax.experimental import pallas as pl


def kernel(image_inputs, image_grid):
    raise NotImplementedError("write your pallas kernel here")


# speedup vs baseline: 1.3046x; 1.3046x over previous
import functools

import jax
import jax.numpy as jnp
import numpy as np
from jax import lax
from jax.experimental import pallas as pl
from jax.experimental.pallas import tpu as pltpu
from jax.experimental.pallas import tpu_sc as plsc

L = 16
NC, NS = 2, 16
NW = NC * NS
B, N = 4, 262144
NPTS = B * N
PPW = NPTS // NW
K = 2048
NCH = PPW // K
NIDX = 128
NG = K // NIDX
VOLSZ = 128 * 128 * 128

CLIP_LO = np.float32(0.001)
CLIP_HI = np.float32(128.0) - np.float32(1.001)

OFFS = (0, 16384, 128, 16512, 1, 16385, 129, 16513)

_mesh = plsc.VectorSubcoreMesh(core_axis_name="c", subcore_axis_name="s")

_scratch = (
    [pltpu.VMEM((K,), jnp.float32) for _ in range(3)]
    + [pltpu.VMEM((K,), jnp.int32) for _ in range(8)]
    + [pltpu.VMEM((K,), jnp.float32) for _ in range(8)]
    + [pltpu.VMEM((K,), jnp.float32) for _ in range(6)]
    + [pltpu.VMEM((K,), jnp.float32)]
    + [pltpu.SemaphoreType.DMA]
)


@functools.partial(
    pl.kernel,
    mesh=_mesh,
    out_type=jax.ShapeDtypeStruct((NPTS,), jnp.float32),
    scratch_types=_scratch,
)
def _warp(vol, gx, gy, gz, out, *refs):
    grid = (gx, gy, gz)
    coords = refs[0:3]
    idx_s = refs[3:11]
    g_s = refs[11:19]
    w_s = refs[19:25]
    o_s = refs[25]
    sem_g = refs[26]

    cid = lax.axis_index("c")
    sid = lax.axis_index("s")
    wid = sid * NC + cid
    base0 = wid * PPW
    vbase = (wid // (NW // B)) * VOLSZ

    def chunk_body(ch, carry):
        base = base0 + ch * K
        for a in range(3):
            pltpu.sync_copy(grid[a].at[pl.ds(base, K)], coords[a])

        def gen(i, c2):
            sl = pl.ds(i * L, L)

            def axis(a):
                t = coords[a][sl] * 128.0
                t = jnp.minimum(jnp.maximum(t, CLIP_LO), CLIP_HI)
                i1 = t.astype(jnp.int32)
                f1 = i1.astype(jnp.float32)
                w = t - f1
                up = jnp.where(w > 0.0, 1.0, 0.0).astype(jnp.float32)
                w2 = (f1 + up) - t
                return i1, w, w2

            ix, wx, wx2 = axis(0)
            iy, wy, wy2 = axis(1)
            iz, wz, wz2 = axis(2)
            ibase = ix * 16384 + iy * 128 + iz + vbase
            for c in range(8):
                idx_s[c][sl] = ibase + OFFS[c]
            for a, w in enumerate((wx, wx2, wy, wy2, wz, wz2)):
                w_s[a][sl] = w
            return c2

        lax.fori_loop(0, K // L, gen, 0)

        copies = []
        for c in range(8):
            for j in range(NG):
                copies.append(
                    pltpu.async_copy(
                        vol.at[idx_s[c].at[pl.ds(j * NIDX, NIDX)]],
                        g_s[c].at[pl.ds(j * NIDX, NIDX)],
                        sem_g,
                    )
                )
        for cp in copies:
            cp.wait()

        def blend(i, c2):
            sl = pl.ds(i * L, L)
            wx = w_s[0][sl]
            wx2 = w_s[1][sl]
            wy = w_s[2][sl]
            wy2 = w_s[3][sl]
            wz = w_s[4][sl]
            wz2 = w_s[5][sl]
            lx1 = g_s[1][sl] * wx + g_s[0][sl] * wx2
            lx2 = g_s[3][sl] * wx + g_s[2][sl] * wx2
            ly1 = lx2 * wy + lx1 * wy2
            lx1b = g_s[5][sl] * wx + g_s[4][sl] * wx2
            lx2b = g_s[7][sl] * wx + g_s[6][sl] * wx2
            ly2 = lx2b * wy + lx1b * wy2
            o_s[sl] = ly2 * wz + ly1 * wz2
            return c2

        lax.fori_loop(0, K // L, blend, 0)
        pltpu.sync_copy(o_s, out.at[pl.ds(base, K)])
        return carry

    lax.fori_loop(0, NCH, chunk_body, 0)


def kernel(image_inputs, image_grid):
    vol = image_inputs.reshape(-1)
    grid_t = jnp.transpose(image_grid, (2, 0, 1)).reshape(3, NPTS)
    out = _warp(vol, grid_t[0], grid_t[1], grid_t[2])
    return out.reshape(B, N, 1)
